# Initial kernel scaffold; baseline (speedup 1.0000x reference)
import jax
import jax.numpy as jnp
from jax.experimental import pallas as pl


def kernel(q, k, v, edges, edge_index, Wq, bq, Wk, bk, Wv, bv, Wp, bp, Ww1, bw1, Ww2, bw2):
    raise NotImplementedError("write your pallas kernel here")


# speedup vs baseline: 2.8042x; 2.8042x over previous
import functools

import jax
import jax.numpy as jnp
from jax import lax
from jax.experimental import pallas as pl
from jax.experimental.pallas import tpu as pltpu
from jax.experimental.pallas import tpu_sc as plsc

N = 10000
E = 320000
D = 128
DE = 16
DS = 16
NC = 2
NS = 16
NW = NC * NS
EPW = E // NW
BLK = 80
NBLK = EPW // BLK
RCHUNK = 80
NRCHUNK = N // RCHUNK

_sc_mesh = plsc.VectorSubcoreMesh(core_axis_name="c", subcore_axis_name="s")


def _proj_body(qb, kb, vb, wqt, bq, wkt, bk, wvt, bv, oq, ok, ov):
    oq[...] = jnp.dot(qb[...], wqt[...], preferred_element_type=jnp.float32) + bq[...]
    ok[...] = jnp.dot(kb[...], wkt[...], preferred_element_type=jnp.float32) + bk[...]
    ov[...] = jnp.dot(vb[...], wvt[...], preferred_element_type=jnp.float32) + bv[...]


def _project(q, k, v, WqT, bq, WkT, bk, WvT, bv):
    BN = 1000
    grid = (N // BN,)
    row = lambda i: (i, 0)
    fixed = lambda i: (0, 0)
    return pl.pallas_call(
        _proj_body,
        grid=grid,
        in_specs=[
            pl.BlockSpec((BN, D), row),
            pl.BlockSpec((BN, D), row),
            pl.BlockSpec((BN, D), row),
            pl.BlockSpec((D, D), fixed),
            pl.BlockSpec((1, D), fixed),
            pl.BlockSpec((D, D), fixed),
            pl.BlockSpec((1, D), fixed),
            pl.BlockSpec((D, D), fixed),
            pl.BlockSpec((1, D), fixed),
        ],
        out_specs=[
            pl.BlockSpec((BN, D), row),
            pl.BlockSpec((BN, D), row),
            pl.BlockSpec((BN, D), row),
        ],
        out_shape=[jax.ShapeDtypeStruct((N, D), jnp.float32)] * 3,
    )(q, k, v, WqT, bq, WkT, bk, WvT, bv)


def _gather_sub_body(xk_hbm, xq_hbm, src_hbm, dst_hbm, dd_hbm,
                     idx_s, idx_d, rows_k, rows_q, dd_v, sem1, sem2, sem3):
    wid = lax.axis_index("s") * NC + lax.axis_index("c")
    base = wid * EPW

    def block(b, carry):
        off = base + b * BLK
        pltpu.sync_copy(src_hbm.at[pl.ds(off, BLK)], idx_s)
        pltpu.sync_copy(dst_hbm.at[pl.ds(off, BLK)], idx_d)
        cp1 = pltpu.async_copy(xk_hbm.at[idx_s], rows_k, sem1)
        cp2 = pltpu.async_copy(xq_hbm.at[idx_d], rows_q, sem2)
        cp1.wait()
        cp2.wait()

        def edge(e, c2):
            for c in range(D // 16):
                sl = pl.ds(c * 16, 16)
                dd_v[e, sl] = rows_k[e, sl] - rows_q[e, sl]
            return c2

        lax.fori_loop(0, BLK, edge, 0)
        cp3 = pltpu.async_copy(dd_v, dd_hbm.at[pl.ds(off, BLK)], sem3)
        cp3.wait()
        return carry

    lax.fori_loop(0, NBLK, block, 0)


def _gather_sub(xk, xq, src, dst):
    f = pl.kernel(
        _gather_sub_body,
        out_type=jax.ShapeDtypeStruct((E, D), jnp.float32),
        mesh=_sc_mesh,
        scratch_types=[
            pltpu.VMEM((BLK,), jnp.int32),
            pltpu.VMEM((BLK,), jnp.int32),
            pltpu.VMEM((BLK, D), jnp.float32),
            pltpu.VMEM((BLK, D), jnp.float32),
            pltpu.VMEM((BLK, D), jnp.float32),
            pltpu.SemaphoreType.DMA,
            pltpu.SemaphoreType.DMA,
            pltpu.SemaphoreType.DMA,
        ],
    )
    return f(xk, xq, src, dst)


def _mlp_body(dd, eb, w1t, bw1, w2t, bw2, wpr, bp, w_out, wp_out):
    p_r = jnp.sum(eb[...] * wpr[...], axis=1, keepdims=True) + bp[...]
    a = jnp.maximum(dd[...] + p_r, 0.0)
    h = jnp.dot(a, w1t[...], preferred_element_type=jnp.float32) + bw1[...]
    h = jnp.maximum(h, 0.0)
    g = jnp.dot(h, w2t[...], preferred_element_type=jnp.float32) + bw2[...]
    m = jnp.max(g, axis=1, keepdims=True)
    ex = jnp.exp(g - m)
    wgt = ex / jnp.sum(ex, axis=1, keepdims=True)
    w_out[...] = wgt
    wp_out[...] = wgt * p_r


def _mlp(dd, edges, W1T, bw1, W2T, bw2, WpRow, bp):
    BE = 2000
    grid = (E // BE,)
    row = lambda i: (i, 0)
    fixed = lambda i: (0, 0)
    return pl.pallas_call(
        _mlp_body,
        grid=grid,
        in_specs=[
            pl.BlockSpec((BE, D), row),
            pl.BlockSpec((BE, DE), row),
            pl.BlockSpec((D, DS), fixed),
            pl.BlockSpec((1, DS), fixed),
            pl.BlockSpec((DS, DS), fixed),
            pl.BlockSpec((1, DS), fixed),
            pl.BlockSpec((1, DE), fixed),
            pl.BlockSpec((1, 1), fixed),
        ],
        out_specs=[
            pl.BlockSpec((BE, DS), row),
            pl.BlockSpec((BE, DS), row),
        ],
        out_shape=[jax.ShapeDtypeStruct((E, DS), jnp.float32)] * 2,
    )(dd, edges, W1T, bw1, W2T, bw2, WpRow, bp)


def _msg_scatter_body(xv_hbm, src_hbm, dst_hbm, w_hbm, wp_hbm, out_hbm,
                      idx_s, idx_d, rows_v, w_v, wp_v, msg_v, acc,
                      sem1, sem2, sem3):
    cid = lax.axis_index("c")
    sid = lax.axis_index("s")
    wid = sid * NC + cid
    base = wid * EPW

    def zrow(e, c2):
        for c in range(D // 16):
            msg_v[e, pl.ds(c * 16, 16)] = jnp.zeros((16,), jnp.float32)
        return c2

    lax.fori_loop(0, BLK, zrow, 0)

    def _zero_chunk(j):
        pltpu.sync_copy(msg_v, acc.at[pl.ds(j * RCHUNK, RCHUNK)])

    for m in range(8):
        j = sid + NS * m
        pl.when(j < NRCHUNK)(functools.partial(_zero_chunk, j))
    plsc.subcore_barrier()

    def block(b, carry):
        off = base + b * BLK
        pltpu.sync_copy(src_hbm.at[pl.ds(off, BLK)], idx_s)
        pltpu.sync_copy(dst_hbm.at[pl.ds(off, BLK)], idx_d)
        cp1 = pltpu.async_copy(xv_hbm.at[idx_s], rows_v, sem1)
        pltpu.sync_copy(w_hbm.at[pl.ds(off, BLK)], w_v)
        pltpu.sync_copy(wp_hbm.at[pl.ds(off, BLK)], wp_v)
        cp1.wait()

        def edge(e, c2):
            wv = w_v[e, :]
            wpv = wp_v[e, :]
            for c in range(D // 16):
                sl = pl.ds(c * 16, 16)
                msg_v[e, sl] = rows_v[e, sl] * wv + wpv
            return c2

        lax.fori_loop(0, BLK, edge, 0)
        pltpu.sync_copy(msg_v, acc.at[idx_d], add=True)
        return carry

    lax.fori_loop(0, NBLK, block, 0)
    plsc.subcore_barrier()

    def _write_chunk(j):
        sl = pl.ds(j * RCHUNK, RCHUNK)
        pltpu.sync_copy(acc.at[sl], out_hbm.at[cid, sl])

    for m in range(8):
        j = sid + NS * m
        pl.when(j < NRCHUNK)(functools.partial(_write_chunk, j))


def _msg_scatter(xv, src, dst, w, wp):
    f = pl.kernel(
        _msg_scatter_body,
        out_type=jax.ShapeDtypeStruct((NC, N, D), jnp.float32),
        mesh=_sc_mesh,
        scratch_types=[
            pltpu.VMEM((BLK,), jnp.int32),
            pltpu.VMEM((BLK,), jnp.int32),
            pltpu.VMEM((BLK, D), jnp.float32),
            pltpu.VMEM((BLK, DS), jnp.float32),
            pltpu.VMEM((BLK, DS), jnp.float32),
            pltpu.VMEM((BLK, D), jnp.float32),
            pltpu.VMEM_SHARED((N, D), jnp.float32),
            pltpu.SemaphoreType.DMA,
            pltpu.SemaphoreType.DMA,
            pltpu.SemaphoreType.DMA,
        ],
    )
    return f(xv, src, dst, w, wp)


def _sum_body(p, o):
    o[...] = p[0] + p[1]


def _sum_partials(partials):
    BN = 1000
    return pl.pallas_call(
        _sum_body,
        grid=(N // BN,),
        in_specs=[pl.BlockSpec((NC, BN, D), lambda i: (0, i, 0))],
        out_specs=pl.BlockSpec((BN, D), lambda i: (i, 0)),
        out_shape=jax.ShapeDtypeStruct((N, D), jnp.float32),
    )(partials)


def kernel(q, k, v, edges, edge_index, Wq, bq, Wk, bk, Wv, bv, Wp, bp,
           Ww1, bw1, Ww2, bw2):
    dst = edge_index[:, 0]
    src = edge_index[:, 1]
    xq, xk, xv = _project(
        q, k, v,
        Wq.T, bq.reshape(1, D),
        Wk.T, bk.reshape(1, D),
        Wv.T, bv.reshape(1, D),
    )
    dd = _gather_sub(xk, xq, src, dst)
    w, wp = _mlp(
        dd, edges,
        Ww1.T, bw1.reshape(1, DS),
        Ww2.T, bw2.reshape(1, DS),
        Wp.reshape(1, DE), bp.reshape(1, 1),
    )
    partials = _msg_scatter(xv, src, dst, w, wp)
    return _sum_partials(partials)


# speedup vs baseline: 4.0570x; 1.4468x over previous
import functools

import jax
import jax.numpy as jnp
from jax import lax
from jax.experimental import pallas as pl
from jax.experimental.pallas import tpu as pltpu
from jax.experimental.pallas import tpu_sc as plsc

N = 10000
E = 320000
D = 128
DE = 16
DS = 16
NC = 2
NS = 16
NW = NC * NS
EPW = E // NW
BLK = 80
NBLK = EPW // BLK
BLKC = 16
NBLKC = EPW // BLKC
RCHUNK = BLKC
NRCHUNK = N // RCHUNK
WCH = 80
NWCH = N // WCH

_sc_mesh = plsc.VectorSubcoreMesh(core_axis_name="c", subcore_axis_name="s")


def _proj_body(qb, kb, vb, wqt, bq, wkt, bk, wvt, bv, oq, ok, ov):
    oq[...] = jnp.dot(qb[...], wqt[...], preferred_element_type=jnp.float32) + bq[...]
    ok[...] = jnp.dot(kb[...], wkt[...], preferred_element_type=jnp.float32) + bk[...]
    ov[...] = jnp.dot(vb[...], wvt[...], preferred_element_type=jnp.float32) + bv[...]


def _project(q, k, v, WqT, bq, WkT, bk, WvT, bv):
    BN = 1000
    grid = (N // BN,)
    row = lambda i: (i, 0)
    fixed = lambda i: (0, 0)
    return pl.pallas_call(
        _proj_body,
        grid=grid,
        in_specs=[
            pl.BlockSpec((BN, D), row),
            pl.BlockSpec((BN, D), row),
            pl.BlockSpec((BN, D), row),
            pl.BlockSpec((D, D), fixed),
            pl.BlockSpec((1, D), fixed),
            pl.BlockSpec((D, D), fixed),
            pl.BlockSpec((1, D), fixed),
            pl.BlockSpec((D, D), fixed),
            pl.BlockSpec((1, D), fixed),
        ],
        out_specs=[
            pl.BlockSpec((BN, D), row),
            pl.BlockSpec((BN, D), row),
            pl.BlockSpec((BN, D), row),
        ],
        out_shape=[jax.ShapeDtypeStruct((N, D), jnp.float32)] * 3,
    )(q, k, v, WqT, bq, WkT, bk, WvT, bv)


UN = 4


def _sub_block(rk, rq, ddv):
    def edge(i, c2):
        e0 = i * UN
        for u in range(UN):
            e = e0 + u
            for c in range(D // 16):
                sl = pl.ds(c * 16, 16)
                ddv[e, sl] = rk[e, sl] - rq[e, sl]
        return c2

    lax.fori_loop(0, BLK // UN, edge, 0)


def _gather_sub_body(xk_hbm, xq_hbm, src_hbm, dst_hbm, dd_hbm,
                     idx_s, idx_d, rk0, rq0, rk1, rq1, dd0, dd1,
                     sk0, sq0, sk1, sq1, sw0, sw1):
    wid = lax.axis_index("s") * NC + lax.axis_index("c")
    base = wid * EPW

    pltpu.sync_copy(src_hbm.at[pl.ds(base, EPW)], idx_s)
    pltpu.sync_copy(dst_hbm.at[pl.ds(base, EPW)], idx_d)

    def issue(b, rk, rq, sk, sq):
        pltpu.async_copy(xk_hbm.at[idx_s.at[pl.ds(b * BLK, BLK)]], rk, sk)
        pltpu.async_copy(xq_hbm.at[idx_d.at[pl.ds(b * BLK, BLK)]], rq, sq)

    def wait(b, rk, rq, sk, sq):
        pltpu.make_async_copy(
            xk_hbm.at[idx_s.at[pl.ds(b * BLK, BLK)]], rk, sk).wait()
        pltpu.make_async_copy(
            xq_hbm.at[idx_d.at[pl.ds(b * BLK, BLK)]], rq, sq).wait()

    def out_at(b):
        return dd_hbm.at[pl.ds(base + b * BLK, BLK)]

    issue(0, rk0, rq0, sk0, sq0)

    def body(b2, carry):
        b = 2 * b2
        issue(b + 1, rk1, rq1, sk1, sq1)
        wait(b, rk0, rq0, sk0, sq0)
        pl.when(b2 > 0)(
            lambda: pltpu.make_async_copy(dd0, out_at(b - 2), sw0).wait())
        _sub_block(rk0, rq0, dd0)
        pltpu.async_copy(dd0, out_at(b), sw0)
        issue(b + 2, rk0, rq0, sk0, sq0)
        wait(b + 1, rk1, rq1, sk1, sq1)
        pl.when(b2 > 0)(
            lambda: pltpu.make_async_copy(dd1, out_at(b - 1), sw1).wait())
        _sub_block(rk1, rq1, dd1)
        pltpu.async_copy(dd1, out_at(b + 1), sw1)
        return carry

    lax.fori_loop(0, (NBLK - 1) // 2, body, 0)

    bl = NBLK - 1
    wait(bl, rk0, rq0, sk0, sq0)
    pltpu.make_async_copy(dd0, out_at(bl - 2), sw0).wait()
    _sub_block(rk0, rq0, dd0)
    pltpu.async_copy(dd0, out_at(bl), sw0)
    pltpu.make_async_copy(dd1, out_at(bl - 1), sw1).wait()
    pltpu.make_async_copy(dd0, out_at(bl), sw0).wait()


def _gather_sub(xk, xq, src3, dst3):
    f = pl.kernel(
        _gather_sub_body,
        out_type=jax.ShapeDtypeStruct((E, D), jnp.float32),
        mesh=_sc_mesh,
        scratch_types=[
            pltpu.VMEM((EPW,), jnp.int32),
            pltpu.VMEM((EPW,), jnp.int32),
            pltpu.VMEM((BLK, D), jnp.float32),
            pltpu.VMEM((BLK, D), jnp.float32),
            pltpu.VMEM((BLK, D), jnp.float32),
            pltpu.VMEM((BLK, D), jnp.float32),
            pltpu.VMEM((BLK, D), jnp.float32),
            pltpu.VMEM((BLK, D), jnp.float32),
            pltpu.SemaphoreType.DMA,
            pltpu.SemaphoreType.DMA,
            pltpu.SemaphoreType.DMA,
            pltpu.SemaphoreType.DMA,
            pltpu.SemaphoreType.DMA,
            pltpu.SemaphoreType.DMA,
        ],
    )
    return f(xk, xq, src3, dst3)


def _mlp_body(dd, eb, w1t, bw1, w2t, bw2, wpr, bp, w_out, wp_out):
    p_r = jnp.sum(eb[...] * wpr[...], axis=1, keepdims=True) + bp[...]
    a = jnp.maximum(dd[...] + p_r, 0.0)
    h = jnp.dot(a, w1t[...], preferred_element_type=jnp.float32) + bw1[...]
    h = jnp.maximum(h, 0.0)
    g = jnp.dot(h, w2t[...], preferred_element_type=jnp.float32) + bw2[...]
    m = jnp.max(g, axis=1, keepdims=True)
    ex = jnp.exp(g - m)
    wgt = ex / jnp.sum(ex, axis=1, keepdims=True)
    w_out[...] = wgt
    wp_out[...] = wgt * p_r


def _mlp(dd, edges, W1T, bw1, W2T, bw2, WpRow, bp):
    BE = 2000
    grid = (E // BE,)
    row = lambda i: (i, 0)
    fixed = lambda i: (0, 0)
    return pl.pallas_call(
        _mlp_body,
        grid=grid,
        in_specs=[
            pl.BlockSpec((BE, D), row),
            pl.BlockSpec((BE, DE), row),
            pl.BlockSpec((D, DS), fixed),
            pl.BlockSpec((1, DS), fixed),
            pl.BlockSpec((DS, DS), fixed),
            pl.BlockSpec((1, DS), fixed),
            pl.BlockSpec((1, DE), fixed),
            pl.BlockSpec((1, 1), fixed),
        ],
        out_specs=[
            pl.BlockSpec((BE, DS), row),
            pl.BlockSpec((BE, DS), row),
        ],
        out_shape=[jax.ShapeDtypeStruct((E, DS), jnp.float32)] * 2,
    )(dd, edges, W1T, bw1, W2T, bw2, WpRow, bp)


def _msg_block(rv, w_v, wp_v, msg):
    def grp(i, c2):
        e0 = i * UN
        for u in range(UN):
            e = e0 + u
            wv = w_v[e, :]
            wpv = wp_v[e, :]
            for c in range(D // 16):
                sl = pl.ds(c * 16, 16)
                msg[e, sl] = rv[e, sl] * wv + wpv
        return c2

    lax.fori_loop(0, BLKC // UN, grp, 0)


def _msg_scatter_body(xv_hbm, src_hbm, dst_hbm, w_hbm, wp_hbm, out_hbm,
                      idx_s, idx_d, rv0, rv1, w0, w1, wp0, wp1, m0, m1, acc,
                      sv0, sv1, sw0, sw1, sp0, sp1, ss0, ss1):
    cid = lax.axis_index("c")
    sid = lax.axis_index("s")
    wid = sid * NC + cid
    base = wid * EPW

    pltpu.sync_copy(src_hbm.at[pl.ds(base, EPW)], idx_s)
    pltpu.sync_copy(dst_hbm.at[pl.ds(base, EPW)], idx_d)

    def sidx(b):
        return idx_s.at[pl.ds(b * BLKC, BLKC)]

    def didx(b):
        return idx_d.at[pl.ds(b * BLKC, BLKC)]

    def issue(b, rv, w_v, wp_v, sv, sw, sp):
        off = base + b * BLKC
        pltpu.async_copy(xv_hbm.at[sidx(b)], rv, sv)
        pltpu.async_copy(w_hbm.at[pl.ds(off, BLKC)], w_v, sw)
        pltpu.async_copy(wp_hbm.at[pl.ds(off, BLKC)], wp_v, sp)

    def wait(b, rv, w_v, wp_v, sv, sw, sp):
        off = base + b * BLKC
        pltpu.make_async_copy(xv_hbm.at[sidx(b)], rv, sv).wait()
        pltpu.make_async_copy(w_hbm.at[pl.ds(off, BLKC)], w_v, sw).wait()
        pltpu.make_async_copy(wp_hbm.at[pl.ds(off, BLKC)], wp_v, sp).wait()

    issue(0, rv0, w0, wp0, sv0, sw0, sp0)

    def zrow(i, c2):
        for u in range(UN):
            for c in range(D // 16):
                m0[i * UN + u, pl.ds(c * 16, 16)] = jnp.zeros((16,), jnp.float32)
        return c2

    lax.fori_loop(0, BLKC // UN, zrow, 0)

    def _zero_chunk(j):
        pltpu.sync_copy(m0, acc.at[pl.ds(j * RCHUNK, RCHUNK)])

    for m in range(40):
        j = sid + NS * m
        pl.when(j < NRCHUNK)(functools.partial(_zero_chunk, j))
    plsc.subcore_barrier()

    def body(b2, carry):
        b = 2 * b2
        issue(b + 1, rv1, w1, wp1, sv1, sw1, sp1)
        wait(b, rv0, w0, wp0, sv0, sw0, sp0)
        pl.when(b2 > 0)(
            lambda: pltpu.make_async_copy(m0, acc.at[didx(b)], ss0).wait())
        _msg_block(rv0, w0, wp0, m0)
        pltpu.async_copy(m0, acc.at[didx(b)], ss0, add=True)
        issue(b + 2, rv0, w0, wp0, sv0, sw0, sp0)
        wait(b + 1, rv1, w1, wp1, sv1, sw1, sp1)
        pl.when(b2 > 0)(
            lambda: pltpu.make_async_copy(m1, acc.at[didx(b)], ss1).wait())
        _msg_block(rv1, w1, wp1, m1)
        pltpu.async_copy(m1, acc.at[didx(b + 1)], ss1, add=True)
        return carry

    lax.fori_loop(0, (NBLKC - 1) // 2, body, 0)

    bl = NBLKC - 1
    wait(bl, rv0, w0, wp0, sv0, sw0, sp0)
    pltpu.make_async_copy(m0, acc.at[didx(bl)], ss0).wait()
    _msg_block(rv0, w0, wp0, m0)
    pltpu.async_copy(m0, acc.at[didx(bl)], ss0, add=True)
    pltpu.make_async_copy(m1, acc.at[didx(bl)], ss1).wait()
    pltpu.make_async_copy(m0, acc.at[didx(bl)], ss0).wait()

    plsc.subcore_barrier()

    def _write_chunk(j):
        sl = pl.ds(j * WCH, WCH)
        pltpu.sync_copy(acc.at[sl], out_hbm.at[cid, sl])

    for m in range(8):
        j = sid + NS * m
        pl.when(j < NWCH)(functools.partial(_write_chunk, j))


def _msg_scatter(xv, src, dst, w, wp):
    f = pl.kernel(
        _msg_scatter_body,
        out_type=jax.ShapeDtypeStruct((NC, N, D), jnp.float32),
        mesh=_sc_mesh,
        scratch_types=[
            pltpu.VMEM((EPW,), jnp.int32),
            pltpu.VMEM((EPW,), jnp.int32),
            pltpu.VMEM((BLKC, D), jnp.float32),
            pltpu.VMEM((BLKC, D), jnp.float32),
            pltpu.VMEM((BLKC, DS), jnp.float32),
            pltpu.VMEM((BLKC, DS), jnp.float32),
            pltpu.VMEM((BLKC, DS), jnp.float32),
            pltpu.VMEM((BLKC, DS), jnp.float32),
            pltpu.VMEM((BLKC, D), jnp.float32),
            pltpu.VMEM((BLKC, D), jnp.float32),
            pltpu.VMEM_SHARED((N, D), jnp.float32),
            pltpu.SemaphoreType.DMA,
            pltpu.SemaphoreType.DMA,
            pltpu.SemaphoreType.DMA,
            pltpu.SemaphoreType.DMA,
            pltpu.SemaphoreType.DMA,
            pltpu.SemaphoreType.DMA,
            pltpu.SemaphoreType.DMA,
            pltpu.SemaphoreType.DMA,
        ],
    )
    return f(xv, src, dst, w, wp)


def _sum_body(p, o):
    o[...] = p[0] + p[1]


def _sum_partials(partials):
    BN = 1000
    return pl.pallas_call(
        _sum_body,
        grid=(N // BN,),
        in_specs=[pl.BlockSpec((NC, BN, D), lambda i: (0, i, 0))],
        out_specs=pl.BlockSpec((BN, D), lambda i: (i, 0)),
        out_shape=jax.ShapeDtypeStruct((N, D), jnp.float32),
    )(partials)


def kernel(q, k, v, edges, edge_index, Wq, bq, Wk, bk, Wv, bv, Wp, bp,
           Ww1, bw1, Ww2, bw2):
    dst = edge_index[:, 0]
    src = edge_index[:, 1]
    xq, xk, xv = _project(
        q, k, v,
        Wq.T, bq.reshape(1, D),
        Wk.T, bk.reshape(1, D),
        Wv.T, bv.reshape(1, D),
    )
    dd = _gather_sub(xk, xq, src, dst)
    w, wp = _mlp(
        dd, edges,
        Ww1.T, bw1.reshape(1, DS),
        Ww2.T, bw2.reshape(1, DS),
        Wp.reshape(1, DE), bp.reshape(1, 1),
    )
    partials = _msg_scatter(xv, src, dst, w, wp)
    return _sum_partials(partials)


# speedup vs baseline: 5.1870x; 1.2785x over previous
import functools

import jax
import jax.numpy as jnp
from jax import lax
from jax.experimental import pallas as pl
from jax.experimental.pallas import tpu as pltpu
from jax.experimental.pallas import tpu_sc as plsc

N = 10000
E = 320000
D = 128
DE = 16
DS = 16
NC = 2
NS = 16
NW = NC * NS
EPW = E // NW
BLK = 80
NBLK = EPW // BLK
BLKC = 40
NBLKC = EPW // BLKC
RCHUNK = BLKC
NRCHUNK = N // RCHUNK
WCH = 80
NWCH = N // WCH

_sc_mesh = plsc.VectorSubcoreMesh(core_axis_name="c", subcore_axis_name="s")


def _proj_body(qb, kb, vb, wqt, bq, wkt, bk, wvt, bv, oq, ok, ov):
    oq[...] = jnp.dot(qb[...], wqt[...], preferred_element_type=jnp.float32) + bq[...]
    ok[...] = jnp.dot(kb[...], wkt[...], preferred_element_type=jnp.float32) + bk[...]
    ov[...] = jnp.dot(vb[...], wvt[...], preferred_element_type=jnp.float32) + bv[...]


def _project(q, k, v, WqT, bq, WkT, bk, WvT, bv):
    BN = 1000
    grid = (N // BN,)
    row = lambda i: (i, 0)
    fixed = lambda i: (0, 0)
    return pl.pallas_call(
        _proj_body,
        grid=grid,
        in_specs=[
            pl.BlockSpec((BN, D), row),
            pl.BlockSpec((BN, D), row),
            pl.BlockSpec((BN, D), row),
            pl.BlockSpec((D, D), fixed),
            pl.BlockSpec((1, D), fixed),
            pl.BlockSpec((D, D), fixed),
            pl.BlockSpec((1, D), fixed),
            pl.BlockSpec((D, D), fixed),
            pl.BlockSpec((1, D), fixed),
        ],
        out_specs=[
            pl.BlockSpec((BN, D), row),
            pl.BlockSpec((BN, D), row),
            pl.BlockSpec((BN, D), row),
        ],
        out_shape=[jax.ShapeDtypeStruct((N, D), jnp.float32)] * 3,
    )(q, k, v, WqT, bq, WkT, bk, WvT, bv)


UN = 4


def _sub_block(rk, rq, ddv):
    def edge(i, c2):
        e0 = i * UN
        for u in range(UN):
            e = e0 + u
            for c in range(D // 16):
                sl = pl.ds(c * 16, 16)
                ddv[e, sl] = rk[e, sl] - rq[e, sl]
        return c2

    lax.fori_loop(0, BLK // UN, edge, 0)


def _gather_sub_body(xk_hbm, xq_hbm, src_hbm, dst_hbm, dd_hbm,
                     idx_s, idx_d, rk0, rq0, rk1, rq1, dd0, dd1,
                     sk0, sq0, sk1, sq1, sw0, sw1):
    wid = lax.axis_index("s") * NC + lax.axis_index("c")
    base = wid * EPW

    pltpu.sync_copy(src_hbm.at[pl.ds(base, EPW)], idx_s)
    pltpu.sync_copy(dst_hbm.at[pl.ds(base, EPW)], idx_d)

    def issue(b, rk, rq, sk, sq):
        pltpu.async_copy(xk_hbm.at[idx_s.at[pl.ds(b * BLK, BLK)]], rk, sk)
        pltpu.async_copy(xq_hbm.at[idx_d.at[pl.ds(b * BLK, BLK)]], rq, sq)

    def wait(b, rk, rq, sk, sq):
        pltpu.make_async_copy(
            xk_hbm.at[idx_s.at[pl.ds(b * BLK, BLK)]], rk, sk).wait()
        pltpu.make_async_copy(
            xq_hbm.at[idx_d.at[pl.ds(b * BLK, BLK)]], rq, sq).wait()

    def out_at(b):
        return dd_hbm.at[pl.ds(base + b * BLK, BLK)]

    issue(0, rk0, rq0, sk0, sq0)

    def body(b2, carry):
        b = 2 * b2
        issue(b + 1, rk1, rq1, sk1, sq1)
        wait(b, rk0, rq0, sk0, sq0)
        pl.when(b2 > 0)(
            lambda: pltpu.make_async_copy(dd0, out_at(b - 2), sw0).wait())
        _sub_block(rk0, rq0, dd0)
        pltpu.async_copy(dd0, out_at(b), sw0)
        issue(b + 2, rk0, rq0, sk0, sq0)
        wait(b + 1, rk1, rq1, sk1, sq1)
        pl.when(b2 > 0)(
            lambda: pltpu.make_async_copy(dd1, out_at(b - 1), sw1).wait())
        _sub_block(rk1, rq1, dd1)
        pltpu.async_copy(dd1, out_at(b + 1), sw1)
        return carry

    lax.fori_loop(0, (NBLK - 1) // 2, body, 0)

    bl = NBLK - 1
    wait(bl, rk0, rq0, sk0, sq0)
    pltpu.make_async_copy(dd0, out_at(bl - 2), sw0).wait()
    _sub_block(rk0, rq0, dd0)
    pltpu.async_copy(dd0, out_at(bl), sw0)
    pltpu.make_async_copy(dd1, out_at(bl - 1), sw1).wait()
    pltpu.make_async_copy(dd0, out_at(bl), sw0).wait()


def _gather_sub(xk, xq, src3, dst3):
    f = pl.kernel(
        _gather_sub_body,
        out_type=jax.ShapeDtypeStruct((E, D), jnp.float32),
        mesh=_sc_mesh,
        scratch_types=[
            pltpu.VMEM((EPW,), jnp.int32),
            pltpu.VMEM((EPW,), jnp.int32),
            pltpu.VMEM((BLK, D), jnp.float32),
            pltpu.VMEM((BLK, D), jnp.float32),
            pltpu.VMEM((BLK, D), jnp.float32),
            pltpu.VMEM((BLK, D), jnp.float32),
            pltpu.VMEM((BLK, D), jnp.float32),
            pltpu.VMEM((BLK, D), jnp.float32),
            pltpu.SemaphoreType.DMA,
            pltpu.SemaphoreType.DMA,
            pltpu.SemaphoreType.DMA,
            pltpu.SemaphoreType.DMA,
            pltpu.SemaphoreType.DMA,
            pltpu.SemaphoreType.DMA,
        ],
    )
    return f(xk, xq, src3, dst3)


def _mlp_body(dd, eb, w1t, bw1, w2t, bw2, wpr, bp, wwp_out):
    p_r = jnp.sum(eb[...] * wpr[...], axis=1, keepdims=True) + bp[...]
    a = jnp.maximum(dd[...] + p_r, 0.0)
    h = jnp.dot(a, w1t[...], preferred_element_type=jnp.float32) + bw1[...]
    h = jnp.maximum(h, 0.0)
    g = jnp.dot(h, w2t[...], preferred_element_type=jnp.float32) + bw2[...]
    m = jnp.max(g, axis=1, keepdims=True)
    ex = jnp.exp(g - m)
    wgt = ex / jnp.sum(ex, axis=1, keepdims=True)
    wwp_out[...] = jnp.concatenate([wgt, wgt * p_r], axis=1)


def _mlp(dd, edges, W1T, bw1, W2T, bw2, WpRow, bp):
    BE = 4000
    grid = (E // BE,)
    row = lambda i: (i, 0)
    fixed = lambda i: (0, 0)
    return pl.pallas_call(
        _mlp_body,
        grid=grid,
        in_specs=[
            pl.BlockSpec((BE, D), row),
            pl.BlockSpec((BE, DE), row),
            pl.BlockSpec((D, DS), fixed),
            pl.BlockSpec((1, DS), fixed),
            pl.BlockSpec((DS, DS), fixed),
            pl.BlockSpec((1, DS), fixed),
            pl.BlockSpec((1, DE), fixed),
            pl.BlockSpec((1, 1), fixed),
        ],
        out_specs=pl.BlockSpec((BE, 2 * DS), row),
        out_shape=jax.ShapeDtypeStruct((E, 2 * DS), jnp.float32),
    )(dd, edges, W1T, bw1, W2T, bw2, WpRow, bp)


def _msg_block(rv, wwp, msg):
    def grp(i, c2):
        e0 = i * UN
        for u in range(UN):
            e = e0 + u
            wv = wwp[e, pl.ds(0, 16)]
            wpv = wwp[e, pl.ds(16, 16)]
            for c in range(D // 16):
                sl = pl.ds(c * 16, 16)
                msg[e, sl] = rv[e, sl] * wv + wpv
        return c2

    lax.fori_loop(0, BLKC // UN, grp, 0)


def _msg_scatter_body(xv_hbm, src_hbm, dst_hbm, wwp_hbm, out_hbm,
                      idx_s, idx_d, rv0, rv1, w0, w1, m0, m1, acc,
                      sv0, sv1, sw0, sw1, ss0, ss1):
    cid = lax.axis_index("c")
    sid = lax.axis_index("s")
    wid = sid * NC + cid
    base = wid * EPW

    pltpu.sync_copy(src_hbm.at[pl.ds(base, EPW)], idx_s)
    pltpu.sync_copy(dst_hbm.at[pl.ds(base, EPW)], idx_d)

    def sidx(b):
        return idx_s.at[pl.ds(b * BLKC, BLKC)]

    def didx(b):
        return idx_d.at[pl.ds(b * BLKC, BLKC)]

    def issue(b, rv, wwp, sv, sw):
        off = base + b * BLKC
        pltpu.async_copy(xv_hbm.at[sidx(b)], rv, sv)
        pltpu.async_copy(wwp_hbm.at[pl.ds(off, BLKC)], wwp, sw)

    def wait(b, rv, wwp, sv, sw):
        off = base + b * BLKC
        pltpu.make_async_copy(xv_hbm.at[sidx(b)], rv, sv).wait()
        pltpu.make_async_copy(wwp_hbm.at[pl.ds(off, BLKC)], wwp, sw).wait()

    issue(0, rv0, w0, sv0, sw0)

    def zrow(i, c2):
        for u in range(UN):
            for c in range(D // 16):
                m0[i * UN + u, pl.ds(c * 16, 16)] = jnp.zeros((16,), jnp.float32)
        return c2

    lax.fori_loop(0, BLKC // UN, zrow, 0)

    def _zero_chunk(j):
        pltpu.sync_copy(m0, acc.at[pl.ds(j * RCHUNK, RCHUNK)])

    for m in range(16):
        j = sid + NS * m
        pl.when(j < NRCHUNK)(functools.partial(_zero_chunk, j))
    plsc.subcore_barrier()

    def body(b2, carry):
        b = 2 * b2
        issue(b + 1, rv1, w1, sv1, sw1)
        wait(b, rv0, w0, sv0, sw0)
        pl.when(b2 > 0)(
            lambda: pltpu.make_async_copy(m0, acc.at[didx(b)], ss0).wait())
        _msg_block(rv0, w0, m0)
        pltpu.async_copy(m0, acc.at[didx(b)], ss0, add=True)
        issue(b + 2, rv0, w0, sv0, sw0)
        wait(b + 1, rv1, w1, sv1, sw1)
        pl.when(b2 > 0)(
            lambda: pltpu.make_async_copy(m1, acc.at[didx(b)], ss1).wait())
        _msg_block(rv1, w1, m1)
        pltpu.async_copy(m1, acc.at[didx(b + 1)], ss1, add=True)
        return carry

    lax.fori_loop(0, (NBLKC - 2) // 2, body, 0)

    b = NBLKC - 2
    issue(b + 1, rv1, w1, sv1, sw1)
    wait(b, rv0, w0, sv0, sw0)
    pltpu.make_async_copy(m0, acc.at[didx(b)], ss0).wait()
    _msg_block(rv0, w0, m0)
    pltpu.async_copy(m0, acc.at[didx(b)], ss0, add=True)
    wait(b + 1, rv1, w1, sv1, sw1)
    pltpu.make_async_copy(m1, acc.at[didx(b)], ss1).wait()
    _msg_block(rv1, w1, m1)
    pltpu.async_copy(m1, acc.at[didx(b + 1)], ss1, add=True)
    pltpu.make_async_copy(m0, acc.at[didx(b)], ss0).wait()
    pltpu.make_async_copy(m1, acc.at[didx(b)], ss1).wait()

    plsc.subcore_barrier()

    def _write_chunk(j):
        sl = pl.ds(j * WCH, WCH)
        pltpu.sync_copy(acc.at[sl], out_hbm.at[cid, sl])

    for m in range(8):
        j = sid + NS * m
        pl.when(j < NWCH)(functools.partial(_write_chunk, j))


def _msg_scatter(xv, src, dst, wwp):
    f = pl.kernel(
        _msg_scatter_body,
        out_type=jax.ShapeDtypeStruct((NC, N, D), jnp.float32),
        mesh=_sc_mesh,
        scratch_types=[
            pltpu.VMEM((EPW,), jnp.int32),
            pltpu.VMEM((EPW,), jnp.int32),
            pltpu.VMEM((BLKC, D), jnp.float32),
            pltpu.VMEM((BLKC, D), jnp.float32),
            pltpu.VMEM((BLKC, 2 * DS), jnp.float32),
            pltpu.VMEM((BLKC, 2 * DS), jnp.float32),
            pltpu.VMEM((BLKC, D), jnp.float32),
            pltpu.VMEM((BLKC, D), jnp.float32),
            pltpu.VMEM_SHARED((N, D), jnp.float32),
            pltpu.SemaphoreType.DMA,
            pltpu.SemaphoreType.DMA,
            pltpu.SemaphoreType.DMA,
            pltpu.SemaphoreType.DMA,
            pltpu.SemaphoreType.DMA,
            pltpu.SemaphoreType.DMA,
        ],
    )
    return f(xv, src, dst, wwp)


def _sum_body(p, o):
    o[...] = p[0] + p[1]


def _sum_partials(partials):
    BN = 1000
    return pl.pallas_call(
        _sum_body,
        grid=(N // BN,),
        in_specs=[pl.BlockSpec((NC, BN, D), lambda i: (0, i, 0))],
        out_specs=pl.BlockSpec((BN, D), lambda i: (i, 0)),
        out_shape=jax.ShapeDtypeStruct((N, D), jnp.float32),
    )(partials)


def kernel(q, k, v, edges, edge_index, Wq, bq, Wk, bk, Wv, bv, Wp, bp,
           Ww1, bw1, Ww2, bw2):
    dst = edge_index[:, 0]
    src = edge_index[:, 1]
    xq, xk, xv = _project(
        q, k, v,
        Wq.T, bq.reshape(1, D),
        Wk.T, bk.reshape(1, D),
        Wv.T, bv.reshape(1, D),
    )
    dd = _gather_sub(xk, xq, src, dst)
    wwp = _mlp(
        dd, edges,
        Ww1.T, bw1.reshape(1, DS),
        Ww2.T, bw2.reshape(1, DS),
        Wp.reshape(1, DE), bp.reshape(1, 1),
    )
    partials = _msg_scatter(xv, src, dst, wwp)
    return _sum_partials(partials)


# speedup vs baseline: 5.7900x; 1.1163x over previous
import functools

import jax
import jax.numpy as jnp
from jax import lax
from jax.experimental import pallas as pl
from jax.experimental.pallas import tpu as pltpu
from jax.experimental.pallas import tpu_sc as plsc

N = 10000
E = 320000
D = 128
DE = 16
DS = 16
NC = 2
NS = 16
NW = NC * NS
NH = 2
EH = E // NH
EPW = EH // NW
BLK = 40
NBLK = EPW // BLK
BLKC = 40
NBLKC = EPW // BLKC
RCHUNK = BLKC
NRCHUNK = N // RCHUNK
WCH = 80
NWCH = N // WCH

_sc_mesh = plsc.VectorSubcoreMesh(core_axis_name="c", subcore_axis_name="s")


def _proj_body(qb, kb, vb, wqt, bq, wkt, bk, wvt, bv, oq, ok, ov):
    oq[...] = jnp.dot(qb[...], wqt[...], preferred_element_type=jnp.float32) + bq[...]
    ok[...] = jnp.dot(kb[...], wkt[...], preferred_element_type=jnp.float32) + bk[...]
    ov[...] = jnp.dot(vb[...], wvt[...], preferred_element_type=jnp.float32) + bv[...]


def _project(q, k, v, WqT, bq, WkT, bk, WvT, bv):
    BN = 1000
    grid = (N // BN,)
    row = lambda i: (i, 0)
    fixed = lambda i: (0, 0)
    return pl.pallas_call(
        _proj_body,
        grid=grid,
        in_specs=[
            pl.BlockSpec((BN, D), row),
            pl.BlockSpec((BN, D), row),
            pl.BlockSpec((BN, D), row),
            pl.BlockSpec((D, D), fixed),
            pl.BlockSpec((1, D), fixed),
            pl.BlockSpec((D, D), fixed),
            pl.BlockSpec((1, D), fixed),
            pl.BlockSpec((D, D), fixed),
            pl.BlockSpec((1, D), fixed),
        ],
        out_specs=[
            pl.BlockSpec((BN, D), row),
            pl.BlockSpec((BN, D), row),
            pl.BlockSpec((BN, D), row),
        ],
        out_shape=[jax.ShapeDtypeStruct((N, D), jnp.float32)] * 3,
    )(q, k, v, WqT, bq, WkT, bk, WvT, bv)


UN = 4


def _sub_block(rk, rq, ddv):
    def edge(i, c2):
        e0 = i * UN
        for u in range(UN):
            e = e0 + u
            for c in range(D // 16):
                sl = pl.ds(c * 16, 16)
                ddv[e, sl] = rk[e, sl] - rq[e, sl]
        return c2

    lax.fori_loop(0, BLK // UN, edge, 0)


def _gather_sub_body(xk_hbm, xq_hbm, src_hbm, dst_hbm, dd_hbm,
                     idx_s, idx_d, rk0, rq0, rk1, rq1, dd0, dd1,
                     sk0, sq0, sk1, sq1, sw0, sw1):
    wid = lax.axis_index("s") * NC + lax.axis_index("c")
    base = wid * EPW

    pltpu.sync_copy(src_hbm.at[pl.ds(base, EPW)], idx_s)
    pltpu.sync_copy(dst_hbm.at[pl.ds(base, EPW)], idx_d)

    def issue(b, rk, rq, sk, sq):
        pltpu.async_copy(xk_hbm.at[idx_s.at[pl.ds(b * BLK, BLK)]], rk, sk)
        pltpu.async_copy(xq_hbm.at[idx_d.at[pl.ds(b * BLK, BLK)]], rq, sq)

    def wait(b, rk, rq, sk, sq):
        pltpu.make_async_copy(
            xk_hbm.at[idx_s.at[pl.ds(b * BLK, BLK)]], rk, sk).wait()
        pltpu.make_async_copy(
            xq_hbm.at[idx_d.at[pl.ds(b * BLK, BLK)]], rq, sq).wait()

    def out_at(b):
        return dd_hbm.at[pl.ds(base + b * BLK, BLK)]

    issue(0, rk0, rq0, sk0, sq0)

    def body(b2, carry):
        b = 2 * b2
        issue(b + 1, rk1, rq1, sk1, sq1)
        wait(b, rk0, rq0, sk0, sq0)
        pl.when(b2 > 0)(
            lambda: pltpu.make_async_copy(dd0, out_at(b - 2), sw0).wait())
        _sub_block(rk0, rq0, dd0)
        pltpu.async_copy(dd0, out_at(b), sw0)
        issue(b + 2, rk0, rq0, sk0, sq0)
        wait(b + 1, rk1, rq1, sk1, sq1)
        pl.when(b2 > 0)(
            lambda: pltpu.make_async_copy(dd1, out_at(b - 1), sw1).wait())
        _sub_block(rk1, rq1, dd1)
        pltpu.async_copy(dd1, out_at(b + 1), sw1)
        return carry

    lax.fori_loop(0, (NBLK - 1) // 2, body, 0)

    bl = NBLK - 1
    wait(bl, rk0, rq0, sk0, sq0)
    pltpu.make_async_copy(dd0, out_at(bl - 2), sw0).wait()
    _sub_block(rk0, rq0, dd0)
    pltpu.async_copy(dd0, out_at(bl), sw0)
    pltpu.make_async_copy(dd1, out_at(bl - 1), sw1).wait()
    pltpu.make_async_copy(dd0, out_at(bl), sw0).wait()


def _gather_sub(xk, xq, src3, dst3):
    f = pl.kernel(
        _gather_sub_body,
        out_type=jax.ShapeDtypeStruct((EH, D), jnp.float32),
        mesh=_sc_mesh,
        scratch_types=[
            pltpu.VMEM((EPW,), jnp.int32),
            pltpu.VMEM((EPW,), jnp.int32),
            pltpu.VMEM((BLK, D), jnp.float32),
            pltpu.VMEM((BLK, D), jnp.float32),
            pltpu.VMEM((BLK, D), jnp.float32),
            pltpu.VMEM((BLK, D), jnp.float32),
            pltpu.VMEM((BLK, D), jnp.float32),
            pltpu.VMEM((BLK, D), jnp.float32),
            pltpu.SemaphoreType.DMA,
            pltpu.SemaphoreType.DMA,
            pltpu.SemaphoreType.DMA,
            pltpu.SemaphoreType.DMA,
            pltpu.SemaphoreType.DMA,
            pltpu.SemaphoreType.DMA,
        ],
    )
    return f(xk, xq, src3, dst3)


def _mlp_body(dd, eb, w1t, bw1, w2t, bw2, wpr, bp, wwp_out):
    p_r = jnp.sum(eb[...] * wpr[...], axis=1, keepdims=True) + bp[...]
    a = jnp.maximum(dd[...] + p_r, 0.0)
    h = jnp.dot(a, w1t[...], preferred_element_type=jnp.float32) + bw1[...]
    h = jnp.maximum(h, 0.0)
    g = jnp.dot(h, w2t[...], preferred_element_type=jnp.float32) + bw2[...]
    m = jnp.max(g, axis=1, keepdims=True)
    ex = jnp.exp(g - m)
    wgt = ex / jnp.sum(ex, axis=1, keepdims=True)
    wwp_out[...] = jnp.concatenate([wgt, wgt * p_r], axis=1)


def _mlp(dd, edges, W1T, bw1, W2T, bw2, WpRow, bp):
    BE = 4000
    grid = (EH // BE,)
    row = lambda i: (i, 0)
    fixed = lambda i: (0, 0)
    return pl.pallas_call(
        _mlp_body,
        grid=grid,
        in_specs=[
            pl.BlockSpec((BE, D), row),
            pl.BlockSpec((BE, DE), row),
            pl.BlockSpec((D, DS), fixed),
            pl.BlockSpec((1, DS), fixed),
            pl.BlockSpec((DS, DS), fixed),
            pl.BlockSpec((1, DS), fixed),
            pl.BlockSpec((1, DE), fixed),
            pl.BlockSpec((1, 1), fixed),
        ],
        out_specs=pl.BlockSpec((BE, 2 * DS), row),
        out_shape=jax.ShapeDtypeStruct((EH, 2 * DS), jnp.float32),
    )(dd, edges, W1T, bw1, W2T, bw2, WpRow, bp)


def _msg_block(rv, wwp, msg):
    def grp(i, c2):
        e0 = i * UN
        for u in range(UN):
            e = e0 + u
            wv = wwp[e, pl.ds(0, 16)]
            wpv = wwp[e, pl.ds(16, 16)]
            for c in range(D // 16):
                sl = pl.ds(c * 16, 16)
                msg[e, sl] = rv[e, sl] * wv + wpv
        return c2

    lax.fori_loop(0, BLKC // UN, grp, 0)


def _msg_scatter_body(xv_hbm, src_hbm, dst_hbm, wwp_hbm, out_hbm,
                      idx_s, idx_d, rv0, rv1, w0, w1, m0, m1, acc,
                      sv0, sv1, sw0, sw1, ss0, ss1):
    cid = lax.axis_index("c")
    sid = lax.axis_index("s")
    wid = sid * NC + cid
    base = wid * EPW

    pltpu.sync_copy(src_hbm.at[pl.ds(base, EPW)], idx_s)
    pltpu.sync_copy(dst_hbm.at[pl.ds(base, EPW)], idx_d)

    def sidx(b):
        return idx_s.at[pl.ds(b * BLKC, BLKC)]

    def didx(b):
        return idx_d.at[pl.ds(b * BLKC, BLKC)]

    def issue(b, rv, wwp, sv, sw):
        off = base + b * BLKC
        pltpu.async_copy(xv_hbm.at[sidx(b)], rv, sv)
        pltpu.async_copy(wwp_hbm.at[pl.ds(off, BLKC)], wwp, sw)

    def wait(b, rv, wwp, sv, sw):
        off = base + b * BLKC
        pltpu.make_async_copy(xv_hbm.at[sidx(b)], rv, sv).wait()
        pltpu.make_async_copy(wwp_hbm.at[pl.ds(off, BLKC)], wwp, sw).wait()

    issue(0, rv0, w0, sv0, sw0)

    def zrow(i, c2):
        for u in range(UN):
            for c in range(D // 16):
                m0[i * UN + u, pl.ds(c * 16, 16)] = jnp.zeros((16,), jnp.float32)
        return c2

    lax.fori_loop(0, BLKC // UN, zrow, 0)

    def _zero_chunk(j):
        pltpu.sync_copy(m0, acc.at[pl.ds(j * RCHUNK, RCHUNK)])

    for m in range(16):
        j = sid + NS * m
        pl.when(j < NRCHUNK)(functools.partial(_zero_chunk, j))
    plsc.subcore_barrier()

    def body(b2, carry):
        b = 2 * b2
        issue(b + 1, rv1, w1, sv1, sw1)
        wait(b, rv0, w0, sv0, sw0)
        pl.when(b2 > 0)(
            lambda: pltpu.make_async_copy(m0, acc.at[didx(b)], ss0).wait())
        _msg_block(rv0, w0, m0)
        pltpu.async_copy(m0, acc.at[didx(b)], ss0, add=True)
        issue(b + 2, rv0, w0, sv0, sw0)
        wait(b + 1, rv1, w1, sv1, sw1)
        pl.when(b2 > 0)(
            lambda: pltpu.make_async_copy(m1, acc.at[didx(b)], ss1).wait())
        _msg_block(rv1, w1, m1)
        pltpu.async_copy(m1, acc.at[didx(b + 1)], ss1, add=True)
        return carry

    lax.fori_loop(0, (NBLKC - 1) // 2, body, 0)

    bl = NBLKC - 1
    wait(bl, rv0, w0, sv0, sw0)
    pltpu.make_async_copy(m0, acc.at[didx(bl)], ss0).wait()
    _msg_block(rv0, w0, m0)
    pltpu.async_copy(m0, acc.at[didx(bl)], ss0, add=True)
    pltpu.make_async_copy(m1, acc.at[didx(bl)], ss1).wait()
    pltpu.make_async_copy(m0, acc.at[didx(bl)], ss0).wait()

    plsc.subcore_barrier()

    def _write_chunk(j):
        sl = pl.ds(j * WCH, WCH)
        pltpu.sync_copy(acc.at[sl], out_hbm.at[cid, sl])

    for m in range(8):
        j = sid + NS * m
        pl.when(j < NWCH)(functools.partial(_write_chunk, j))


def _msg_scatter(xv, src, dst, wwp):
    f = pl.kernel(
        _msg_scatter_body,
        out_type=jax.ShapeDtypeStruct((NC, N, D), jnp.float32),
        mesh=_sc_mesh,
        scratch_types=[
            pltpu.VMEM((EPW,), jnp.int32),
            pltpu.VMEM((EPW,), jnp.int32),
            pltpu.VMEM((BLKC, D), jnp.float32),
            pltpu.VMEM((BLKC, D), jnp.float32),
            pltpu.VMEM((BLKC, 2 * DS), jnp.float32),
            pltpu.VMEM((BLKC, 2 * DS), jnp.float32),
            pltpu.VMEM((BLKC, D), jnp.float32),
            pltpu.VMEM((BLKC, D), jnp.float32),
            pltpu.VMEM_SHARED((N, D), jnp.float32),
            pltpu.SemaphoreType.DMA,
            pltpu.SemaphoreType.DMA,
            pltpu.SemaphoreType.DMA,
            pltpu.SemaphoreType.DMA,
            pltpu.SemaphoreType.DMA,
            pltpu.SemaphoreType.DMA,
        ],
    )
    return f(xv, src, dst, wwp)


def _sum_body(pa, pb, o):
    o[...] = (pa[0] + pa[1]) + (pb[0] + pb[1])


def _sum_partials(pa, pb):
    BN = 1000
    return pl.pallas_call(
        _sum_body,
        grid=(N // BN,),
        in_specs=[
            pl.BlockSpec((NC, BN, D), lambda i: (0, i, 0)),
            pl.BlockSpec((NC, BN, D), lambda i: (0, i, 0)),
        ],
        out_specs=pl.BlockSpec((BN, D), lambda i: (i, 0)),
        out_shape=jax.ShapeDtypeStruct((N, D), jnp.float32),
    )(pa, pb)


def kernel(q, k, v, edges, edge_index, Wq, bq, Wk, bk, Wv, bv, Wp, bp,
           Ww1, bw1, Ww2, bw2):
    dst = edge_index[:, 0]
    src = edge_index[:, 1]
    xq, xk, xv = _project(
        q, k, v,
        Wq.T, bq.reshape(1, D),
        Wk.T, bk.reshape(1, D),
        Wv.T, bv.reshape(1, D),
    )
    w1t = Ww1.T
    bw1r = bw1.reshape(1, DS)
    w2t = Ww2.T
    bw2r = bw2.reshape(1, DS)
    wpr = Wp.reshape(1, DE)
    bpr = bp.reshape(1, 1)
    parts = []
    for h in range(NH):
        sl = slice(h * EH, (h + 1) * EH)
        src_h, dst_h, edges_h = src[sl], dst[sl], edges[sl]
        dd = _gather_sub(xk, xq, src_h, dst_h)
        wwp = _mlp(dd, edges_h, w1t, bw1r, w2t, bw2r, wpr, bpr)
        parts.append(_msg_scatter(xv, src_h, dst_h, wwp))
    return _sum_partials(parts[0], parts[1])


# speedup vs baseline: 5.8633x; 1.0127x over previous
import functools

import jax
import jax.numpy as jnp
from jax import lax
from jax.experimental import pallas as pl
from jax.experimental.pallas import tpu as pltpu
from jax.experimental.pallas import tpu_sc as plsc

N = 10000
E = 320000
D = 128
DE = 16
DS = 16
NC = 2
NS = 16
NW = NC * NS
NH = 2
EH = E // NH
EPW = EH // NW
BLK = 40
NBLK = EPW // BLK
BLKC = 40
NBLKC = EPW // BLKC
RCHUNK = BLKC
NRCHUNK = N // RCHUNK
WCH = 80
NWCH = N // WCH

_sc_mesh = plsc.VectorSubcoreMesh(core_axis_name="c", subcore_axis_name="s")


def _proj_body(qb, kb, vb, wqt, bq, wkt, bk, wvt, bv, oq, ok, ov):
    oq[...] = jnp.dot(qb[...], wqt[...], preferred_element_type=jnp.float32) + bq[...]
    ok[...] = jnp.dot(kb[...], wkt[...], preferred_element_type=jnp.float32) + bk[...]
    ov[...] = jnp.dot(vb[...], wvt[...], preferred_element_type=jnp.float32) + bv[...]


def _project(q, k, v, WqT, bq, WkT, bk, WvT, bv):
    BN = 1000
    grid = (N // BN,)
    row = lambda i: (i, 0)
    fixed = lambda i: (0, 0)
    return pl.pallas_call(
        _proj_body,
        grid=grid,
        in_specs=[
            pl.BlockSpec((BN, D), row),
            pl.BlockSpec((BN, D), row),
            pl.BlockSpec((BN, D), row),
            pl.BlockSpec((D, D), fixed),
            pl.BlockSpec((1, D), fixed),
            pl.BlockSpec((D, D), fixed),
            pl.BlockSpec((1, D), fixed),
            pl.BlockSpec((D, D), fixed),
            pl.BlockSpec((1, D), fixed),
        ],
        out_specs=[
            pl.BlockSpec((BN, D), row),
            pl.BlockSpec((BN, D), row),
            pl.BlockSpec((BN, D), row),
        ],
        out_shape=[jax.ShapeDtypeStruct((N, D), jnp.float32)] * 3,
    )(q, k, v, WqT, bq, WkT, bk, WvT, bv)


UN = 8


def _sub_block(rk, rq, ddv):
    def edge(i, c2):
        e0 = i * UN
        for u in range(UN):
            e = e0 + u
            for c in range(D // 16):
                sl = pl.ds(c * 16, 16)
                ddv[e, sl] = rk[e, sl] - rq[e, sl]
        return c2

    lax.fori_loop(0, BLK // UN, edge, 0)


def _gather_sub_body(xk_hbm, xq_hbm, src_hbm, dst_hbm, dd_hbm,
                     idx_s, idx_d, rk0, rq0, rk1, rq1, dd0, dd1,
                     sk0, sq0, sk1, sq1, sw0, sw1):
    wid = lax.axis_index("s") * NC + lax.axis_index("c")
    base = wid * EPW

    pltpu.sync_copy(src_hbm.at[pl.ds(base, EPW)], idx_s)
    pltpu.sync_copy(dst_hbm.at[pl.ds(base, EPW)], idx_d)

    def issue(b, rk, rq, sk, sq):
        pltpu.async_copy(xk_hbm.at[idx_s.at[pl.ds(b * BLK, BLK)]], rk, sk)
        pltpu.async_copy(xq_hbm.at[idx_d.at[pl.ds(b * BLK, BLK)]], rq, sq)

    def wait(b, rk, rq, sk, sq):
        pltpu.make_async_copy(
            xk_hbm.at[idx_s.at[pl.ds(b * BLK, BLK)]], rk, sk).wait()
        pltpu.make_async_copy(
            xq_hbm.at[idx_d.at[pl.ds(b * BLK, BLK)]], rq, sq).wait()

    def out_at(b):
        return dd_hbm.at[pl.ds(base + b * BLK, BLK)]

    issue(0, rk0, rq0, sk0, sq0)

    def body(b2, carry):
        b = 2 * b2
        issue(b + 1, rk1, rq1, sk1, sq1)
        wait(b, rk0, rq0, sk0, sq0)
        pl.when(b2 > 0)(
            lambda: pltpu.make_async_copy(dd0, out_at(b - 2), sw0).wait())
        _sub_block(rk0, rq0, dd0)
        pltpu.async_copy(dd0, out_at(b), sw0)
        issue(b + 2, rk0, rq0, sk0, sq0)
        wait(b + 1, rk1, rq1, sk1, sq1)
        pl.when(b2 > 0)(
            lambda: pltpu.make_async_copy(dd1, out_at(b - 1), sw1).wait())
        _sub_block(rk1, rq1, dd1)
        pltpu.async_copy(dd1, out_at(b + 1), sw1)
        return carry

    lax.fori_loop(0, (NBLK - 1) // 2, body, 0)

    bl = NBLK - 1
    wait(bl, rk0, rq0, sk0, sq0)
    pltpu.make_async_copy(dd0, out_at(bl - 2), sw0).wait()
    _sub_block(rk0, rq0, dd0)
    pltpu.async_copy(dd0, out_at(bl), sw0)
    pltpu.make_async_copy(dd1, out_at(bl - 1), sw1).wait()
    pltpu.make_async_copy(dd0, out_at(bl), sw0).wait()


def _gather_sub(xk, xq, src_h, dst_h):
    f = pl.kernel(
        _gather_sub_body,
        out_type=jax.ShapeDtypeStruct((EH, D), jnp.float32),
        mesh=_sc_mesh,
        scratch_types=[
            pltpu.VMEM((EPW,), jnp.int32),
            pltpu.VMEM((EPW,), jnp.int32),
            pltpu.VMEM((BLK, D), jnp.float32),
            pltpu.VMEM((BLK, D), jnp.float32),
            pltpu.VMEM((BLK, D), jnp.float32),
            pltpu.VMEM((BLK, D), jnp.float32),
            pltpu.VMEM((BLK, D), jnp.float32),
            pltpu.VMEM((BLK, D), jnp.float32),
            pltpu.SemaphoreType.DMA,
            pltpu.SemaphoreType.DMA,
            pltpu.SemaphoreType.DMA,
            pltpu.SemaphoreType.DMA,
            pltpu.SemaphoreType.DMA,
            pltpu.SemaphoreType.DMA,
        ],
    )
    return f(xk, xq, src_h, dst_h)


def _mlp_body(dd, eb, w1t, bw1, w2t, bw2, wpr, bp, wwp_out):
    p_r = jnp.sum(eb[...] * wpr[...], axis=1, keepdims=True) + bp[...]
    a = jnp.maximum(dd[...] + p_r, 0.0)
    h = jnp.dot(a, w1t[...], preferred_element_type=jnp.float32) + bw1[...]
    h = jnp.maximum(h, 0.0)
    g = jnp.dot(h, w2t[...], preferred_element_type=jnp.float32) + bw2[...]
    m = jnp.max(g, axis=1, keepdims=True)
    ex = jnp.exp(g - m)
    wgt = ex / jnp.sum(ex, axis=1, keepdims=True)
    wwp_out[...] = jnp.concatenate([wgt, wgt * p_r], axis=1)


def _mlp(dd, edges, W1T, bw1, W2T, bw2, WpRow, bp):
    BE = 8000
    grid = (EH // BE,)
    row = lambda i: (i, 0)
    fixed = lambda i: (0, 0)
    return pl.pallas_call(
        _mlp_body,
        grid=grid,
        in_specs=[
            pl.BlockSpec((BE, D), row),
            pl.BlockSpec((BE, DE), row),
            pl.BlockSpec((D, DS), fixed),
            pl.BlockSpec((1, DS), fixed),
            pl.BlockSpec((DS, DS), fixed),
            pl.BlockSpec((1, DS), fixed),
            pl.BlockSpec((1, DE), fixed),
            pl.BlockSpec((1, 1), fixed),
        ],
        out_specs=pl.BlockSpec((BE, 2 * DS), row),
        out_shape=jax.ShapeDtypeStruct((EH, 2 * DS), jnp.float32),
    )(dd, edges, W1T, bw1, W2T, bw2, WpRow, bp)


def _msg_block(rv, wwp, msg):
    def grp(i, c2):
        e0 = i * UN
        for u in range(UN):
            e = e0 + u
            wv = wwp[e, pl.ds(0, 16)]
            wpv = wwp[e, pl.ds(16, 16)]
            for c in range(D // 16):
                sl = pl.ds(c * 16, 16)
                msg[e, sl] = rv[e, sl] * wv + wpv
        return c2

    lax.fori_loop(0, BLKC // UN, grp, 0)


def _msg_scatter_body(xv_hbm, src_hbm, dst_hbm, wwp_hbm, out_hbm,
                      idx_s, idx_d, rv0, rv1, w0, w1, m0, m1, acc,
                      sv0, sv1, sw0, sw1, ss0, ss1):
    cid = lax.axis_index("c")
    sid = lax.axis_index("s")
    wid = sid * NC + cid
    base = wid * EPW

    pltpu.sync_copy(src_hbm.at[pl.ds(base, EPW)], idx_s)
    pltpu.sync_copy(dst_hbm.at[pl.ds(base, EPW)], idx_d)

    def sidx(b):
        return idx_s.at[pl.ds(b * BLKC, BLKC)]

    def didx(b):
        return idx_d.at[pl.ds(b * BLKC, BLKC)]

    def issue(b, rv, wwp, sv, sw):
        off = base + b * BLKC
        pltpu.async_copy(xv_hbm.at[sidx(b)], rv, sv)
        pltpu.async_copy(wwp_hbm.at[pl.ds(off, BLKC)], wwp, sw)

    def wait(b, rv, wwp, sv, sw):
        off = base + b * BLKC
        pltpu.make_async_copy(xv_hbm.at[sidx(b)], rv, sv).wait()
        pltpu.make_async_copy(wwp_hbm.at[pl.ds(off, BLKC)], wwp, sw).wait()

    issue(0, rv0, w0, sv0, sw0)

    def zrow(i, c2):
        for u in range(UN):
            for c in range(D // 16):
                m0[i * UN + u, pl.ds(c * 16, 16)] = jnp.zeros((16,), jnp.float32)
        return c2

    lax.fori_loop(0, BLKC // UN, zrow, 0)

    def _zero_chunk(j):
        pltpu.async_copy(m0, acc.at[pl.ds(j * RCHUNK, RCHUNK)], ss1)

    def _zero_drain(j):
        pltpu.make_async_copy(m0, acc.at[pl.ds(j * RCHUNK, RCHUNK)], ss1).wait()

    for m in range(16):
        j = sid + NS * m
        pl.when(j < NRCHUNK)(functools.partial(_zero_chunk, j))
    for m in range(16):
        j = sid + NS * m
        pl.when(j < NRCHUNK)(functools.partial(_zero_drain, j))
    plsc.subcore_barrier()

    def body(b2, carry):
        b = 2 * b2
        issue(b + 1, rv1, w1, sv1, sw1)
        wait(b, rv0, w0, sv0, sw0)
        pl.when(b2 > 0)(
            lambda: pltpu.make_async_copy(m0, acc.at[didx(b)], ss0).wait())
        _msg_block(rv0, w0, m0)
        pltpu.async_copy(m0, acc.at[didx(b)], ss0, add=True)
        issue(b + 2, rv0, w0, sv0, sw0)
        wait(b + 1, rv1, w1, sv1, sw1)
        pl.when(b2 > 0)(
            lambda: pltpu.make_async_copy(m1, acc.at[didx(b)], ss1).wait())
        _msg_block(rv1, w1, m1)
        pltpu.async_copy(m1, acc.at[didx(b + 1)], ss1, add=True)
        return carry

    lax.fori_loop(0, (NBLKC - 1) // 2, body, 0)

    bl = NBLKC - 1
    wait(bl, rv0, w0, sv0, sw0)
    pltpu.make_async_copy(m0, acc.at[didx(bl)], ss0).wait()
    _msg_block(rv0, w0, m0)
    pltpu.async_copy(m0, acc.at[didx(bl)], ss0, add=True)
    pltpu.make_async_copy(m1, acc.at[didx(bl)], ss1).wait()
    pltpu.make_async_copy(m0, acc.at[didx(bl)], ss0).wait()

    plsc.subcore_barrier()

    def _write_chunk(j):
        sl = pl.ds(j * WCH, WCH)
        pltpu.async_copy(acc.at[sl], out_hbm.at[cid, sl], ss0)

    def _write_drain(j):
        sl = pl.ds(j * WCH, WCH)
        pltpu.make_async_copy(acc.at[sl], out_hbm.at[cid, sl], ss0).wait()

    for m in range(8):
        j = sid + NS * m
        pl.when(j < NWCH)(functools.partial(_write_chunk, j))
    for m in range(8):
        j = sid + NS * m
        pl.when(j < NWCH)(functools.partial(_write_drain, j))


def _msg_scatter(xv, src, dst, wwp):
    f = pl.kernel(
        _msg_scatter_body,
        out_type=jax.ShapeDtypeStruct((NC, N, D), jnp.float32),
        mesh=_sc_mesh,
        scratch_types=[
            pltpu.VMEM((EPW,), jnp.int32),
            pltpu.VMEM((EPW,), jnp.int32),
            pltpu.VMEM((BLKC, D), jnp.float32),
            pltpu.VMEM((BLKC, D), jnp.float32),
            pltpu.VMEM((BLKC, 2 * DS), jnp.float32),
            pltpu.VMEM((BLKC, 2 * DS), jnp.float32),
            pltpu.VMEM((BLKC, D), jnp.float32),
            pltpu.VMEM((BLKC, D), jnp.float32),
            pltpu.VMEM_SHARED((N, D), jnp.float32),
            pltpu.SemaphoreType.DMA,
            pltpu.SemaphoreType.DMA,
            pltpu.SemaphoreType.DMA,
            pltpu.SemaphoreType.DMA,
            pltpu.SemaphoreType.DMA,
            pltpu.SemaphoreType.DMA,
        ],
    )
    return f(xv, src, dst, wwp)


def _sum_body(pa, pb, o):
    o[...] = (pa[0] + pa[1]) + (pb[0] + pb[1])


def _sum_partials(pa, pb):
    BN = 1000
    return pl.pallas_call(
        _sum_body,
        grid=(N // BN,),
        in_specs=[
            pl.BlockSpec((NC, BN, D), lambda i: (0, i, 0)),
            pl.BlockSpec((NC, BN, D), lambda i: (0, i, 0)),
        ],
        out_specs=pl.BlockSpec((BN, D), lambda i: (i, 0)),
        out_shape=jax.ShapeDtypeStruct((N, D), jnp.float32),
    )(pa, pb)


def kernel(q, k, v, edges, edge_index, Wq, bq, Wk, bk, Wv, bv, Wp, bp,
           Ww1, bw1, Ww2, bw2):
    dst = edge_index[:, 0]
    src = edge_index[:, 1]
    xq, xk, xv = _project(
        q, k, v,
        Wq.T, bq.reshape(1, D),
        Wk.T, bk.reshape(1, D),
        Wv.T, bv.reshape(1, D),
    )
    w1t = Ww1.T
    bw1r = bw1.reshape(1, DS)
    w2t = Ww2.T
    bw2r = bw2.reshape(1, DS)
    wpr = Wp.reshape(1, DE)
    bpr = bp.reshape(1, 1)
    parts = []
    for h in range(NH):
        sl = slice(h * EH, (h + 1) * EH)
        src_h, dst_h, edges_h = src[sl], dst[sl], edges[sl]
        dd = _gather_sub(xk, xq, src_h, dst_h)
        wwp = _mlp(dd, edges_h, w1t, bw1r, w2t, bw2r, wpr, bpr)
        parts.append(_msg_scatter(xv, src_h, dst_h, wwp))
    return _sum_partials(parts[0], parts[1])


# speedup vs baseline: 6.1715x; 1.0526x over previous
import functools

import jax
import jax.numpy as jnp
from jax import lax
from jax.experimental import pallas as pl
from jax.experimental.pallas import tpu as pltpu
from jax.experimental.pallas import tpu_sc as plsc

N = 10000
E = 320000
D = 128
DE = 16
DS = 16
NC = 2
NS = 16
NW = NC * NS
NH = 2
EH = E // NH
EPW = EH // NW
BLK = 40
NBLK = EPW // BLK
BLKC = 40
NBLKC = EPW // BLKC
RCHUNK = BLKC
NRCHUNK = N // RCHUNK
WCH = 80
NWCH = N // WCH

_sc_mesh = plsc.VectorSubcoreMesh(core_axis_name="c", subcore_axis_name="s")


def _proj_body(qb, kb, vb, wqt, bq, wkt, bk, wvt, bv, oq, ok, ov):
    oq[...] = jnp.dot(qb[...], wqt[...], preferred_element_type=jnp.float32) + bq[...]
    ok[...] = jnp.dot(kb[...], wkt[...], preferred_element_type=jnp.float32) + bk[...]
    ov[...] = jnp.dot(vb[...], wvt[...], preferred_element_type=jnp.float32) + bv[...]


def _project(q, k, v, WqT, bq, WkT, bk, WvT, bv):
    BN = 1000
    grid = (N // BN,)
    row = lambda i: (i, 0)
    fixed = lambda i: (0, 0)
    return pl.pallas_call(
        _proj_body,
        grid=grid,
        in_specs=[
            pl.BlockSpec((BN, D), row),
            pl.BlockSpec((BN, D), row),
            pl.BlockSpec((BN, D), row),
            pl.BlockSpec((D, D), fixed),
            pl.BlockSpec((1, D), fixed),
            pl.BlockSpec((D, D), fixed),
            pl.BlockSpec((1, D), fixed),
            pl.BlockSpec((D, D), fixed),
            pl.BlockSpec((1, D), fixed),
        ],
        out_specs=[
            pl.BlockSpec((BN, D), row),
            pl.BlockSpec((BN, D), row),
            pl.BlockSpec((BN, D), row),
        ],
        out_shape=[jax.ShapeDtypeStruct((N, D), jnp.float32)] * 3,
    )(q, k, v, WqT, bq, WkT, bk, WvT, bv)


UN = 8


def _sub_block(rk, rq, ddv):
    def edge(i, c2):
        e0 = i * UN
        for u in range(UN):
            e = e0 + u
            for c in range(D // 16):
                sl = pl.ds(c * 16, 16)
                ddv[e, sl] = rk[e, sl] - rq[e, sl]
        return c2

    lax.fori_loop(0, BLK // UN, edge, 0)


def _gather_sub_body(xk_hbm, xq_hbm, src_hbm, dst_hbm, dd_hbm,
                     idx_s, idx_d, rk0, rq0, rk1, rq1, dd0, dd1, xk_sp,
                     sk0, sq0, sk1, sq1, sw0, sw1, sst):
    sid = lax.axis_index("s")
    wid = sid * NC + lax.axis_index("c")
    base = wid * EPW

    def _stage(j):
        sl = pl.ds(j * WCH, WCH)
        pltpu.async_copy(xk_hbm.at[sl], xk_sp.at[sl], sst)

    def _stage_drain(j):
        sl = pl.ds(j * WCH, WCH)
        pltpu.make_async_copy(xk_hbm.at[sl], xk_sp.at[sl], sst).wait()

    for m in range(8):
        j = sid + NS * m
        pl.when(j < NWCH)(functools.partial(_stage, j))

    pltpu.sync_copy(src_hbm.at[pl.ds(base, EPW)], idx_s)
    pltpu.sync_copy(dst_hbm.at[pl.ds(base, EPW)], idx_d)

    for m in range(8):
        j = sid + NS * m
        pl.when(j < NWCH)(functools.partial(_stage_drain, j))
    plsc.subcore_barrier()

    def issue(b, rk, rq, sk, sq):
        pltpu.async_copy(xk_sp.at[idx_s.at[pl.ds(b * BLK, BLK)]], rk, sk)
        pltpu.async_copy(xq_hbm.at[idx_d.at[pl.ds(b * BLK, BLK)]], rq, sq)

    def wait(b, rk, rq, sk, sq):
        pltpu.make_async_copy(
            xk_sp.at[idx_s.at[pl.ds(b * BLK, BLK)]], rk, sk).wait()
        pltpu.make_async_copy(
            xq_hbm.at[idx_d.at[pl.ds(b * BLK, BLK)]], rq, sq).wait()

    def out_at(b):
        return dd_hbm.at[pl.ds(base + b * BLK, BLK)]

    issue(0, rk0, rq0, sk0, sq0)

    def body(b2, carry):
        b = 2 * b2
        issue(b + 1, rk1, rq1, sk1, sq1)
        wait(b, rk0, rq0, sk0, sq0)
        pl.when(b2 > 0)(
            lambda: pltpu.make_async_copy(dd0, out_at(b - 2), sw0).wait())
        _sub_block(rk0, rq0, dd0)
        pltpu.async_copy(dd0, out_at(b), sw0)
        issue(b + 2, rk0, rq0, sk0, sq0)
        wait(b + 1, rk1, rq1, sk1, sq1)
        pl.when(b2 > 0)(
            lambda: pltpu.make_async_copy(dd1, out_at(b - 1), sw1).wait())
        _sub_block(rk1, rq1, dd1)
        pltpu.async_copy(dd1, out_at(b + 1), sw1)
        return carry

    lax.fori_loop(0, (NBLK - 1) // 2, body, 0)

    bl = NBLK - 1
    wait(bl, rk0, rq0, sk0, sq0)
    pltpu.make_async_copy(dd0, out_at(bl - 2), sw0).wait()
    _sub_block(rk0, rq0, dd0)
    pltpu.async_copy(dd0, out_at(bl), sw0)
    pltpu.make_async_copy(dd1, out_at(bl - 1), sw1).wait()
    pltpu.make_async_copy(dd0, out_at(bl), sw0).wait()


def _gather_sub(xk, xq, src_h, dst_h):
    f = pl.kernel(
        _gather_sub_body,
        out_type=jax.ShapeDtypeStruct((EH, D), jnp.float32),
        mesh=_sc_mesh,
        scratch_types=[
            pltpu.VMEM((EPW,), jnp.int32),
            pltpu.VMEM((EPW,), jnp.int32),
            pltpu.VMEM((BLK, D), jnp.float32),
            pltpu.VMEM((BLK, D), jnp.float32),
            pltpu.VMEM((BLK, D), jnp.float32),
            pltpu.VMEM((BLK, D), jnp.float32),
            pltpu.VMEM((BLK, D), jnp.float32),
            pltpu.VMEM((BLK, D), jnp.float32),
            pltpu.VMEM_SHARED((N, D), jnp.float32),
            pltpu.SemaphoreType.DMA,
            pltpu.SemaphoreType.DMA,
            pltpu.SemaphoreType.DMA,
            pltpu.SemaphoreType.DMA,
            pltpu.SemaphoreType.DMA,
            pltpu.SemaphoreType.DMA,
            pltpu.SemaphoreType.DMA,
        ],
    )
    return f(xk, xq, src_h, dst_h)


def _mlp_body(dd, eb, w1t, bw1, w2t, bw2, wpr, bp, wwp_out):
    p_r = jnp.sum(eb[...] * wpr[...], axis=1, keepdims=True) + bp[...]
    a = jnp.maximum(dd[...] + p_r, 0.0)
    h = jnp.dot(a, w1t[...], preferred_element_type=jnp.float32) + bw1[...]
    h = jnp.maximum(h, 0.0)
    g = jnp.dot(h, w2t[...], preferred_element_type=jnp.float32) + bw2[...]
    m = jnp.max(g, axis=1, keepdims=True)
    ex = jnp.exp(g - m)
    wgt = ex / jnp.sum(ex, axis=1, keepdims=True)
    wwp_out[...] = jnp.concatenate([wgt, wgt * p_r], axis=1)


def _mlp(dd, edges, W1T, bw1, W2T, bw2, WpRow, bp):
    BE = 8000
    grid = (EH // BE,)
    row = lambda i: (i, 0)
    fixed = lambda i: (0, 0)
    return pl.pallas_call(
        _mlp_body,
        grid=grid,
        in_specs=[
            pl.BlockSpec((BE, D), row),
            pl.BlockSpec((BE, DE), row),
            pl.BlockSpec((D, DS), fixed),
            pl.BlockSpec((1, DS), fixed),
            pl.BlockSpec((DS, DS), fixed),
            pl.BlockSpec((1, DS), fixed),
            pl.BlockSpec((1, DE), fixed),
            pl.BlockSpec((1, 1), fixed),
        ],
        out_specs=pl.BlockSpec((BE, 2 * DS), row),
        out_shape=jax.ShapeDtypeStruct((EH, 2 * DS), jnp.float32),
    )(dd, edges, W1T, bw1, W2T, bw2, WpRow, bp)


def _msg_block(rv, wwp, msg):
    def grp(i, c2):
        e0 = i * UN
        for u in range(UN):
            e = e0 + u
            wv = wwp[e, pl.ds(0, 16)]
            wpv = wwp[e, pl.ds(16, 16)]
            for c in range(D // 16):
                sl = pl.ds(c * 16, 16)
                msg[e, sl] = rv[e, sl] * wv + wpv
        return c2

    lax.fori_loop(0, BLKC // UN, grp, 0)


def _msg_scatter_body(xv_hbm, src_hbm, dst_hbm, wwp_hbm, out_hbm,
                      idx_s, idx_d, rv0, rv1, w0, w1, m0, m1, acc,
                      sv0, sv1, sw0, sw1, ss0, ss1):
    cid = lax.axis_index("c")
    sid = lax.axis_index("s")
    wid = sid * NC + cid
    base = wid * EPW

    pltpu.sync_copy(src_hbm.at[pl.ds(base, EPW)], idx_s)
    pltpu.sync_copy(dst_hbm.at[pl.ds(base, EPW)], idx_d)

    def sidx(b):
        return idx_s.at[pl.ds(b * BLKC, BLKC)]

    def didx(b):
        return idx_d.at[pl.ds(b * BLKC, BLKC)]

    def issue(b, rv, wwp, sv, sw):
        off = base + b * BLKC
        pltpu.async_copy(xv_hbm.at[sidx(b)], rv, sv)
        pltpu.async_copy(wwp_hbm.at[pl.ds(off, BLKC)], wwp, sw)

    def wait(b, rv, wwp, sv, sw):
        off = base + b * BLKC
        pltpu.make_async_copy(xv_hbm.at[sidx(b)], rv, sv).wait()
        pltpu.make_async_copy(wwp_hbm.at[pl.ds(off, BLKC)], wwp, sw).wait()

    issue(0, rv0, w0, sv0, sw0)

    def zrow(i, c2):
        for u in range(UN):
            for c in range(D // 16):
                m0[i * UN + u, pl.ds(c * 16, 16)] = jnp.zeros((16,), jnp.float32)
        return c2

    lax.fori_loop(0, BLKC // UN, zrow, 0)

    def _zero_chunk(j):
        pltpu.async_copy(m0, acc.at[pl.ds(j * RCHUNK, RCHUNK)], ss1)

    def _zero_drain(j):
        pltpu.make_async_copy(m0, acc.at[pl.ds(j * RCHUNK, RCHUNK)], ss1).wait()

    for m in range(16):
        j = sid + NS * m
        pl.when(j < NRCHUNK)(functools.partial(_zero_chunk, j))
    for m in range(16):
        j = sid + NS * m
        pl.when(j < NRCHUNK)(functools.partial(_zero_drain, j))
    plsc.subcore_barrier()

    def body(b2, carry):
        b = 2 * b2
        issue(b + 1, rv1, w1, sv1, sw1)
        wait(b, rv0, w0, sv0, sw0)
        pl.when(b2 > 0)(
            lambda: pltpu.make_async_copy(m0, acc.at[didx(b)], ss0).wait())
        _msg_block(rv0, w0, m0)
        pltpu.async_copy(m0, acc.at[didx(b)], ss0, add=True)
        issue(b + 2, rv0, w0, sv0, sw0)
        wait(b + 1, rv1, w1, sv1, sw1)
        pl.when(b2 > 0)(
            lambda: pltpu.make_async_copy(m1, acc.at[didx(b)], ss1).wait())
        _msg_block(rv1, w1, m1)
        pltpu.async_copy(m1, acc.at[didx(b + 1)], ss1, add=True)
        return carry

    lax.fori_loop(0, (NBLKC - 1) // 2, body, 0)

    bl = NBLKC - 1
    wait(bl, rv0, w0, sv0, sw0)
    pltpu.make_async_copy(m0, acc.at[didx(bl)], ss0).wait()
    _msg_block(rv0, w0, m0)
    pltpu.async_copy(m0, acc.at[didx(bl)], ss0, add=True)
    pltpu.make_async_copy(m1, acc.at[didx(bl)], ss1).wait()
    pltpu.make_async_copy(m0, acc.at[didx(bl)], ss0).wait()

    plsc.subcore_barrier()

    def _write_chunk(j):
        sl = pl.ds(j * WCH, WCH)
        pltpu.async_copy(acc.at[sl], out_hbm.at[cid, sl], ss0)

    def _write_drain(j):
        sl = pl.ds(j * WCH, WCH)
        pltpu.make_async_copy(acc.at[sl], out_hbm.at[cid, sl], ss0).wait()

    for m in range(8):
        j = sid + NS * m
        pl.when(j < NWCH)(functools.partial(_write_chunk, j))
    for m in range(8):
        j = sid + NS * m
        pl.when(j < NWCH)(functools.partial(_write_drain, j))


def _msg_scatter(xv, src, dst, wwp):
    f = pl.kernel(
        _msg_scatter_body,
        out_type=jax.ShapeDtypeStruct((NC, N, D), jnp.float32),
        mesh=_sc_mesh,
        scratch_types=[
            pltpu.VMEM((EPW,), jnp.int32),
            pltpu.VMEM((EPW,), jnp.int32),
            pltpu.VMEM((BLKC, D), jnp.float32),
            pltpu.VMEM((BLKC, D), jnp.float32),
            pltpu.VMEM((BLKC, 2 * DS), jnp.float32),
            pltpu.VMEM((BLKC, 2 * DS), jnp.float32),
            pltpu.VMEM((BLKC, D), jnp.float32),
            pltpu.VMEM((BLKC, D), jnp.float32),
            pltpu.VMEM_SHARED((N, D), jnp.float32),
            pltpu.SemaphoreType.DMA,
            pltpu.SemaphoreType.DMA,
            pltpu.SemaphoreType.DMA,
            pltpu.SemaphoreType.DMA,
            pltpu.SemaphoreType.DMA,
            pltpu.SemaphoreType.DMA,
        ],
    )
    return f(xv, src, dst, wwp)


def _sum_body(pa, pb, o):
    o[...] = (pa[0] + pa[1]) + (pb[0] + pb[1])


def _sum_partials(pa, pb):
    BN = 1000
    return pl.pallas_call(
        _sum_body,
        grid=(N // BN,),
        in_specs=[
            pl.BlockSpec((NC, BN, D), lambda i: (0, i, 0)),
            pl.BlockSpec((NC, BN, D), lambda i: (0, i, 0)),
        ],
        out_specs=pl.BlockSpec((BN, D), lambda i: (i, 0)),
        out_shape=jax.ShapeDtypeStruct((N, D), jnp.float32),
    )(pa, pb)


def kernel(q, k, v, edges, edge_index, Wq, bq, Wk, bk, Wv, bv, Wp, bp,
           Ww1, bw1, Ww2, bw2):
    dst = edge_index[:, 0]
    src = edge_index[:, 1]
    xq, xk, xv = _project(
        q, k, v,
        Wq.T, bq.reshape(1, D),
        Wk.T, bk.reshape(1, D),
        Wv.T, bv.reshape(1, D),
    )
    w1t = Ww1.T
    bw1r = bw1.reshape(1, DS)
    w2t = Ww2.T
    bw2r = bw2.reshape(1, DS)
    wpr = Wp.reshape(1, DE)
    bpr = bp.reshape(1, 1)
    parts = []
    for h in range(NH):
        sl = slice(h * EH, (h + 1) * EH)
        src_h, dst_h, edges_h = src[sl], dst[sl], edges[sl]
        dd = _gather_sub(xk, xq, src_h, dst_h)
        wwp = _mlp(dd, edges_h, w1t, bw1r, w2t, bw2r, wpr, bpr)
        parts.append(_msg_scatter(xv, src_h, dst_h, wwp))
    return _sum_partials(parts[0], parts[1])
